# Initial kernel scaffold; baseline (speedup 1.0000x reference)
import jax
import jax.numpy as jnp
from jax.experimental import pallas as pl


def kernel(hex_xs, hex_ys, terr_ids, mod_flags, unit_type_ids, unit_side_ids, unit_xs, unit_ys, unit_feats, global_feats, terrain_tab, Wmod, posx_tab, posy_tab, unit_type_tab, Wf, bf, side_tab, Wg, bg, end_turn):
    raise NotImplementedError("write your pallas kernel here")


# speedup vs baseline: 1.3011x; 1.3011x over previous
import functools

import jax
import jax.numpy as jnp
from jax import lax
from jax.experimental import pallas as pl
from jax.experimental.pallas import tpu as pltpu
from jax.experimental.pallas import tpu_sc as plsc

N_HEX = 131072
N_UNIT = 32768
D = 128
N_TOK = N_HEX + N_UNIT + 2

NW = 32
NC = 2
HEX_PER_W = N_HEX // NW
UNIT_PER_W = N_UNIT // NW
HP = 1024
SUB = 64


def _sc_encode(hx, hy, ht, hf0, hf1, hf2, ut, ux, uy, P, gf, wg, et,
               px_t, py_t, tm_t, ty_t):
    mesh = plsc.VectorSubcoreMesh(core_axis_name="c", subcore_axis_name="s")

    @functools.partial(
        pl.kernel,
        out_type=jax.ShapeDtypeStruct((N_TOK, D), jnp.float32),
        mesh=mesh,
        compiler_params=pltpu.CompilerParams(needs_layout_passes=False),
        scratch_types=[
            pltpu.VMEM((128, D), jnp.float32),
            pltpu.VMEM((128, D), jnp.float32),
            pltpu.VMEM((128, D), jnp.float32),
            pltpu.VMEM((200, D), jnp.float32),
            pltpu.VMEM((HP,), jnp.int32),
            pltpu.VMEM((HP,), jnp.int32),
            pltpu.VMEM((HP,), jnp.int32),
            pltpu.VMEM((HP,), jnp.float32),
            pltpu.VMEM((HP,), jnp.float32),
            pltpu.VMEM((HP,), jnp.float32),
            pltpu.VMEM((UNIT_PER_W,), jnp.int32),
            pltpu.VMEM((UNIT_PER_W,), jnp.int32),
            pltpu.VMEM((UNIT_PER_W,), jnp.int32),
            pltpu.VMEM((2, SUB, D), jnp.float32),
            pltpu.VMEM((2, SUB, D), jnp.float32),
            pltpu.VMEM((8 * D,), jnp.float32),
            pltpu.VMEM((8 * D,), jnp.float32),
            pltpu.VMEM((2, D), jnp.float32),
            pltpu.SemaphoreType.DMA,
            pltpu.SemaphoreType.DMA,
            pltpu.SemaphoreType.DMA,
            pltpu.SemaphoreType.DMA,
        ],
    )
    def k(hx_h, hy_h, ht_h, f0_h, f1_h, f2_h, ut_h, ux_h, uy_h, p_h,
          gf_h, wg_h, et_h,
          px_h, py_h, tm_h, ty_h, out_h,
          px_v, py_v, tm_v, ty_v, hx_v, hy_v, ht_v, f0_v, f1_v, f2_v,
          ut_v, ux_v, uy_v, p_v, o_v, gf_v, wg_v, g2_v,
          so0, so1, sp0, sp1):
        wid = lax.axis_index("s") * NC + lax.axis_index("c")
        lanes = lax.iota(jnp.int32, 16)
        so = (so0, so1)
        sp = (sp0, sp1)

        pltpu.sync_copy(px_h, px_v)
        pltpu.sync_copy(py_h, py_v)
        pltpu.sync_copy(tm_h, tm_v)
        pltpu.sync_copy(ty_h, ty_v)

        hbase = wid * HEX_PER_W

        @pl.loop(0, HEX_PER_W // HP)
        def _piece(piece):
            pb = hbase + piece * HP
            pltpu.sync_copy(hx_h.at[pl.ds(pb, HP)], hx_v)
            pltpu.sync_copy(hy_h.at[pl.ds(pb, HP)], hy_v)
            pltpu.sync_copy(ht_h.at[pl.ds(pb, HP)], ht_v)
            pltpu.sync_copy(f0_h.at[pl.ds(pb, HP)], f0_v)
            pltpu.sync_copy(f1_h.at[pl.ds(pb, HP)], f1_v)
            pltpu.sync_copy(f2_h.at[pl.ds(pb, HP)], f2_v)

            @pl.loop(0, HP // SUB, step=2)
            def _sub(sub):
                for b in range(2):
                    s = sub + b
                    t = piece * (HP // SUB) + s

                    @pl.when(t >= 2)
                    def _():
                        pltpu.make_async_copy(
                            out_h.at[pl.ds(0, SUB)], o_v.at[b], so[b]).wait()

                    for g in range(4):
                        o = s * SUB + g * 16
                        xs = hx_v[pl.ds(o, 16)]
                        ys = hy_v[pl.ds(o, 16)]
                        ts = ht_v[pl.ds(o, 16)]
                        f0 = f0_v[pl.ds(o, 16)]
                        f1 = f1_v[pl.ds(o, 16)]
                        f2 = f2_v[pl.ds(o, 16)]
                        m = (f0 * 4.0 + f1 * 2.0 + f2).astype(jnp.int32)
                        tmr = ts * 8 + m
                        lg = lanes + g * 16

                        @pl.loop(0, D, unroll=8)
                        def _d(d):
                            dd = jnp.full((16,), d, dtype=jnp.int32)
                            vx = plsc.load_gather(px_v, [xs, dd])
                            vy = plsc.load_gather(py_v, [ys, dd])
                            vt = plsc.load_gather(tm_v, [tmr, dd])
                            plsc.store_scatter(o_v.at[b], [lg, dd],
                                               vx + vy + vt)

                    rowb = pb + s * SUB
                    pltpu.async_copy(o_v.at[b], out_h.at[pl.ds(rowb, SUB)],
                                     so[b])

        for b in range(2):
            pltpu.make_async_copy(out_h.at[pl.ds(0, SUB)], o_v.at[b],
                                  so[b]).wait()

        ubase = wid * UNIT_PER_W
        pltpu.sync_copy(ut_h.at[pl.ds(ubase, UNIT_PER_W)], ut_v)
        pltpu.sync_copy(ux_h.at[pl.ds(ubase, UNIT_PER_W)], ux_v)
        pltpu.sync_copy(uy_h.at[pl.ds(ubase, UNIT_PER_W)], uy_v)

        NSUB = UNIT_PER_W // SUB
        for b in range(2):
            pltpu.async_copy(p_h.at[pl.ds(ubase + b * SUB, SUB)],
                             p_v.at[b], sp[b])

        @pl.loop(0, NSUB, step=2)
        def _usub(sub):
            for b in range(2):
                s = sub + b
                pltpu.make_async_copy(p_h.at[pl.ds(0, SUB)], p_v.at[b],
                                      sp[b]).wait()

                @pl.when(s >= 2)
                def _():
                    pltpu.make_async_copy(
                        out_h.at[pl.ds(0, SUB)], o_v.at[b], so[b]).wait()

                for g in range(4):
                    o = s * SUB + g * 16
                    tv = ut_v[pl.ds(o, 16)]
                    xv = ux_v[pl.ds(o, 16)]
                    yv = uy_v[pl.ds(o, 16)]
                    lg = lanes + g * 16

                    @pl.loop(0, D, unroll=8)
                    def _d(d):
                        dd = jnp.full((16,), d, dtype=jnp.int32)
                        r = (plsc.load_gather(ty_v, [tv, dd])
                             + plsc.load_gather(px_v, [xv, dd])
                             + plsc.load_gather(py_v, [yv, dd])
                             + plsc.load_gather(p_v.at[b], [lg, dd]))
                        plsc.store_scatter(o_v.at[b], [lg, dd], r)

                rowb = ubase + s * SUB
                pltpu.async_copy(o_v.at[b],
                                 out_h.at[pl.ds(N_HEX + rowb, SUB)], so[b])

                @pl.when(s + 2 < NSUB)
                def _():
                    pltpu.async_copy(p_h.at[pl.ds(ubase + (s + 2) * SUB, SUB)],
                                     p_v.at[b], sp[b])

        for b in range(2):
            pltpu.make_async_copy(out_h.at[pl.ds(0, SUB)], o_v.at[b],
                                  so[b]).wait()

        @pl.when(wid == NW - 1)
        def _tail():
            pltpu.sync_copy(gf_h, gf_v)
            pltpu.sync_copy(wg_h, wg_v)
            pltpu.sync_copy(et_h, g2_v.at[pl.ds(1, 1)])
            z16 = jnp.zeros((16,), jnp.int32)
            for dg in range(8):
                acc = jnp.zeros((16,), jnp.float32)
                for kk in range(8):
                    wkr = wg_v[pl.ds(kk * D + dg * 16, 16)]
                    gk = gf_v[pl.ds(kk * D + dg * 16, 16)]
                    acc = acc + gk * wkr
                plsc.store_scatter(g2_v, [z16, dg * 16 + lanes], acc)
            pltpu.sync_copy(g2_v, out_h.at[pl.ds(N_HEX + N_UNIT, 2)])

    return k(hx, hy, ht, hf0, hf1, hf2, ut, ux, uy, P, gf, wg, et,
             px_t, py_t, tm_t, ty_t)


def _tc_proj(X, W):
    PB = 2048

    def body(x_ref, w_ref, o_ref):
        o_ref[...] = jnp.dot(x_ref[...], w_ref[...],
                             preferred_element_type=jnp.float32)

    return pl.pallas_call(
        body,
        grid=(N_UNIT // PB,),
        in_specs=[
            pl.BlockSpec((PB, 16), lambda i: (i, 0)),
            pl.BlockSpec((16, D), lambda i: (0, 0)),
        ],
        out_specs=pl.BlockSpec((PB, D), lambda i: (i, 0)),
        out_shape=jax.ShapeDtypeStruct((N_UNIT, D), jnp.float32),
    )(X, W)


def kernel(hex_xs, hex_ys, terr_ids, mod_flags, unit_type_ids, unit_side_ids,
           unit_xs, unit_ys, unit_feats, global_feats, terrain_tab, Wmod,
           posx_tab, posy_tab, unit_type_tab, Wf, bf, side_tab, Wg, bg,
           end_turn):
    f32 = jnp.float32
    i32 = jnp.int32

    bits = ((jnp.arange(8)[:, None] >> jnp.array([2, 1, 0])[None, :]) & 1
            ).astype(f32)
    modcomb = bits @ Wmod
    terrmod = (terrain_tab[:, None, :] + modcomb[None, :, :]
               ).reshape(16 * 8, D).astype(f32)

    side01 = unit_side_ids.astype(f32)[:, None]
    ones = jnp.ones((N_UNIT, 1), f32)
    zeros = jnp.zeros((N_UNIT, 1), f32)
    X = jnp.concatenate([unit_feats.astype(f32), side01, ones, zeros],
                        axis=1)
    W = jnp.concatenate([
        Wf.astype(f32),
        (side_tab[1] - side_tab[0])[None, :],
        (bf + side_tab[0])[None, :],
        jnp.zeros((1, D), f32),
    ], axis=0)

    P = _tc_proj(X, W)

    gf1 = jnp.concatenate([global_feats[0].astype(f32),
                           jnp.ones((1,), f32), jnp.zeros((1,), f32)])
    gf = jnp.broadcast_to(gf1[:, None], (8, D)).reshape(-1)
    wg = jnp.concatenate([Wg.astype(f32), bg[None, :].astype(f32),
                          jnp.zeros((1, D), f32)], axis=0).reshape(-1)

    out = _sc_encode(
        hex_xs.astype(i32), hex_ys.astype(i32), terr_ids.astype(i32),
        mod_flags[:, 0].astype(f32), mod_flags[:, 1].astype(f32),
        mod_flags[:, 2].astype(f32),
        unit_type_ids.astype(i32), unit_xs.astype(i32), unit_ys.astype(i32),
        P, gf, wg, end_turn.astype(f32)[None, :],
        posx_tab.astype(f32), posy_tab.astype(f32), terrmod,
        unit_type_tab.astype(f32))
    return out[None]


# speedup vs baseline: 5.1505x; 3.9587x over previous
import functools

import jax
import jax.numpy as jnp
from jax import lax
from jax.experimental import pallas as pl
from jax.experimental.pallas import tpu as pltpu
from jax.experimental.pallas import tpu_sc as plsc

N_HEX = 131072
N_UNIT = 32768
D = 128
N_TOK = N_HEX + N_UNIT + 2

NW = 32
NC = 2
HEX_PER_W = N_HEX // NW
UNIT_PER_W = N_UNIT // NW
HP = 1024
SUB = 64


def _sc_encode(hx, hy, ht, hf0, hf1, hf2, ut, ux, uy, P, gf, wg, et,
               px_t, py_t, tm_t, ty_t):
    mesh = plsc.VectorSubcoreMesh(core_axis_name="c", subcore_axis_name="s")

    @functools.partial(
        pl.kernel,
        out_type=jax.ShapeDtypeStruct((N_TOK * D,), jnp.float32),
        mesh=mesh,
        compiler_params=pltpu.CompilerParams(needs_layout_passes=False),
        scratch_types=[
            pltpu.VMEM((128 * D,), jnp.float32),
            pltpu.VMEM((128 * D,), jnp.float32),
            pltpu.VMEM((128 * D,), jnp.float32),
            pltpu.VMEM((200 * D,), jnp.float32),
            pltpu.VMEM((HP,), jnp.int32),
            pltpu.VMEM((HP,), jnp.int32),
            pltpu.VMEM((HP,), jnp.int32),
            pltpu.VMEM((HP,), jnp.float32),
            pltpu.VMEM((HP,), jnp.float32),
            pltpu.VMEM((HP,), jnp.float32),
            pltpu.VMEM((UNIT_PER_W,), jnp.int32),
            pltpu.VMEM((UNIT_PER_W,), jnp.int32),
            pltpu.VMEM((UNIT_PER_W,), jnp.int32),
            pltpu.VMEM((SUB * D,), jnp.float32),
            pltpu.VMEM((SUB * D,), jnp.float32),
            pltpu.VMEM((SUB * D,), jnp.float32),
            pltpu.VMEM((SUB * D,), jnp.float32),
            pltpu.VMEM((8 * D,), jnp.float32),
            pltpu.VMEM((8 * D,), jnp.float32),
            pltpu.VMEM((2 * D,), jnp.float32),
            pltpu.SemaphoreType.DMA,
            pltpu.SemaphoreType.DMA,
            pltpu.SemaphoreType.DMA,
            pltpu.SemaphoreType.DMA,
        ],
    )
    def k(hx_h, hy_h, ht_h, f0_h, f1_h, f2_h, ut_h, ux_h, uy_h, p_h,
          gf_h, wg_h, et_h,
          px_h, py_h, tm_h, ty_h, out_h,
          px_v, py_v, tm_v, ty_v, hx_v, hy_v, ht_v, f0_v, f1_v, f2_v,
          ut_v, ux_v, uy_v, p0_v, p1_v, o0_v, o1_v, gf_v, wg_v, g2_v,
          so0, so1, sp0, sp1):
        wid = lax.axis_index("s") * NC + lax.axis_index("c")
        so = (so0, so1)
        sp = (sp0, sp1)
        ovs = (o0_v, o1_v)
        pvs = (p0_v, p1_v)

        pltpu.sync_copy(px_h, px_v)
        pltpu.sync_copy(py_h, py_v)
        pltpu.sync_copy(tm_h, tm_v)
        pltpu.sync_copy(ty_h, ty_v)

        hbase = wid * HEX_PER_W

        @pl.loop(0, HEX_PER_W // HP)
        def _piece(piece):
            pb = hbase + piece * HP
            pltpu.sync_copy(hx_h.at[pl.ds(pb, HP)], hx_v)
            pltpu.sync_copy(hy_h.at[pl.ds(pb, HP)], hy_v)
            pltpu.sync_copy(ht_h.at[pl.ds(pb, HP)], ht_v)
            pltpu.sync_copy(f0_h.at[pl.ds(pb, HP)], f0_v)
            pltpu.sync_copy(f1_h.at[pl.ds(pb, HP)], f1_v)
            pltpu.sync_copy(f2_h.at[pl.ds(pb, HP)], f2_v)

            @pl.loop(0, HP // SUB, step=2)
            def _sub(sub):
                for b in range(2):
                    s = sub + b
                    t = piece * (HP // SUB) + s

                    @pl.when(t >= 2)
                    def _():
                        pltpu.make_async_copy(
                            out_h.at[pl.ds(0, SUB * D)], ovs[b],
                            so[b]).wait()

                    @pl.loop(0, 4)
                    def _grp(g):
                        o = s * SUB + g * 16
                        xs = hx_v[pl.ds(o, 16)]
                        ys = hy_v[pl.ds(o, 16)]
                        ts = ht_v[pl.ds(o, 16)]
                        f0 = f0_v[pl.ds(o, 16)]
                        f1 = f1_v[pl.ds(o, 16)]
                        f2 = f2_v[pl.ds(o, 16)]
                        m = (f0 * 4.0 + f1 * 2.0 + f2).astype(jnp.int32)
                        bx = xs * D
                        by = ys * D
                        bt = (ts * 8 + m) * D
                        ob = ovs[b]
                        for u in range(16):
                            xo = bx[u]
                            yo = by[u]
                            to = bt[u]
                            lo = (g * 16 + u) * D
                            rx = [px_v[pl.ds(xo + 16 * vi, 16)]
                                  for vi in range(8)]
                            ry = [py_v[pl.ds(yo + 16 * vi, 16)]
                                  for vi in range(8)]
                            rt = [tm_v[pl.ds(to + 16 * vi, 16)]
                                  for vi in range(8)]
                            for vi in range(8):
                                ob[pl.ds(lo + 16 * vi, 16)] = (
                                    rx[vi] + ry[vi] + rt[vi])

                    rowb = pb + s * SUB
                    pltpu.async_copy(ovs[b],
                                     out_h.at[pl.ds(rowb * D, SUB * D)],
                                     so[b])

        for b in range(2):
            pltpu.make_async_copy(out_h.at[pl.ds(0, SUB * D)], ovs[b],
                                  so[b]).wait()

        ubase = wid * UNIT_PER_W
        pltpu.sync_copy(ut_h.at[pl.ds(ubase, UNIT_PER_W)], ut_v)
        pltpu.sync_copy(ux_h.at[pl.ds(ubase, UNIT_PER_W)], ux_v)
        pltpu.sync_copy(uy_h.at[pl.ds(ubase, UNIT_PER_W)], uy_v)

        NSUB = UNIT_PER_W // SUB
        for b in range(2):
            pltpu.async_copy(p_h.at[pl.ds((ubase + b * SUB) * D, SUB * D)],
                             pvs[b], sp[b])

        @pl.loop(0, NSUB, step=2)
        def _usub(sub):
            for b in range(2):
                s = sub + b
                pltpu.make_async_copy(p_h.at[pl.ds(0, SUB * D)], pvs[b],
                                      sp[b]).wait()

                @pl.when(s >= 2)
                def _():
                    pltpu.make_async_copy(
                        out_h.at[pl.ds(0, SUB * D)], ovs[b], so[b]).wait()

                @pl.loop(0, 4)
                def _grp(g):
                    o = s * SUB + g * 16
                    tv = ut_v[pl.ds(o, 16)]
                    xv = ux_v[pl.ds(o, 16)]
                    yv = uy_v[pl.ds(o, 16)]
                    btv = tv * D
                    bxv = xv * D
                    byv = yv * D
                    ob = ovs[b]
                    pb2 = pvs[b]
                    for u in range(16):
                        to = btv[u]
                        xo = bxv[u]
                        yo = byv[u]
                        lo = (g * 16 + u) * D
                        rt = [ty_v[pl.ds(to + 16 * vi, 16)]
                              for vi in range(8)]
                        rx = [px_v[pl.ds(xo + 16 * vi, 16)]
                              for vi in range(8)]
                        ry = [py_v[pl.ds(yo + 16 * vi, 16)]
                              for vi in range(8)]
                        rp = [pb2[pl.ds(lo + 16 * vi, 16)]
                              for vi in range(8)]
                        for vi in range(8):
                            ob[pl.ds(lo + 16 * vi, 16)] = (
                                (rt[vi] + rx[vi]) + (ry[vi] + rp[vi]))

                rowb = ubase + s * SUB
                pltpu.async_copy(ovs[b],
                                 out_h.at[pl.ds((N_HEX + rowb) * D, SUB * D)],
                                 so[b])

                @pl.when(s + 2 < NSUB)
                def _():
                    pltpu.async_copy(
                        p_h.at[pl.ds((ubase + (s + 2) * SUB) * D, SUB * D)],
                        pvs[b], sp[b])

        for b in range(2):
            pltpu.make_async_copy(out_h.at[pl.ds(0, SUB * D)], ovs[b],
                                  so[b]).wait()

        @pl.when(wid == NW - 1)
        def _tail():
            pltpu.sync_copy(gf_h, gf_v)
            pltpu.sync_copy(wg_h, wg_v)
            pltpu.sync_copy(et_h, g2_v.at[pl.ds(D, D)])
            for dg in range(8):
                acc = jnp.zeros((16,), jnp.float32)
                for kk in range(8):
                    wkr = wg_v[pl.ds(kk * D + dg * 16, 16)]
                    gk = gf_v[pl.ds(kk * D + dg * 16, 16)]
                    acc = acc + gk * wkr
                g2_v[pl.ds(dg * 16, 16)] = acc
            pltpu.sync_copy(g2_v,
                            out_h.at[pl.ds((N_HEX + N_UNIT) * D, 2 * D)])

    return k(hx, hy, ht, hf0, hf1, hf2, ut, ux, uy, P, gf, wg, et,
             px_t, py_t, tm_t, ty_t)


def _tc_proj(X, W):
    PB = 2048

    def body(x_ref, w_ref, o_ref):
        o_ref[...] = jnp.dot(x_ref[...], w_ref[...],
                             preferred_element_type=jnp.float32)

    return pl.pallas_call(
        body,
        grid=(N_UNIT // PB,),
        in_specs=[
            pl.BlockSpec((PB, 16), lambda i: (i, 0)),
            pl.BlockSpec((16, D), lambda i: (0, 0)),
        ],
        out_specs=pl.BlockSpec((PB, D), lambda i: (i, 0)),
        out_shape=jax.ShapeDtypeStruct((N_UNIT, D), jnp.float32),
    )(X, W)


def kernel(hex_xs, hex_ys, terr_ids, mod_flags, unit_type_ids, unit_side_ids,
           unit_xs, unit_ys, unit_feats, global_feats, terrain_tab, Wmod,
           posx_tab, posy_tab, unit_type_tab, Wf, bf, side_tab, Wg, bg,
           end_turn):
    f32 = jnp.float32
    i32 = jnp.int32

    bits = ((jnp.arange(8)[:, None] >> jnp.array([2, 1, 0])[None, :]) & 1
            ).astype(f32)
    modcomb = bits @ Wmod
    terrmod = (terrain_tab[:, None, :] + modcomb[None, :, :]
               ).reshape(16 * 8 * D).astype(f32)

    side01 = unit_side_ids.astype(f32)[:, None]
    ones = jnp.ones((N_UNIT, 1), f32)
    zeros = jnp.zeros((N_UNIT, 1), f32)
    X = jnp.concatenate([unit_feats.astype(f32), side01, ones, zeros],
                        axis=1)
    W = jnp.concatenate([
        Wf.astype(f32),
        (side_tab[1] - side_tab[0])[None, :],
        (bf + side_tab[0])[None, :],
        jnp.zeros((1, D), f32),
    ], axis=0)

    P = _tc_proj(X, W)

    gf1 = jnp.concatenate([global_feats[0].astype(f32),
                           jnp.ones((1,), f32), jnp.zeros((1,), f32)])
    gf = jnp.broadcast_to(gf1[:, None], (8, D)).reshape(-1)
    wg = jnp.concatenate([Wg.astype(f32), bg[None, :].astype(f32),
                          jnp.zeros((1, D), f32)], axis=0).reshape(-1)

    out = _sc_encode(
        hex_xs.astype(i32), hex_ys.astype(i32), terr_ids.astype(i32),
        mod_flags[:, 0].astype(f32), mod_flags[:, 1].astype(f32),
        mod_flags[:, 2].astype(f32),
        unit_type_ids.astype(i32), unit_xs.astype(i32), unit_ys.astype(i32),
        P.reshape(-1), gf, wg, end_turn.astype(f32),
        posx_tab.astype(f32).reshape(-1), posy_tab.astype(f32).reshape(-1),
        terrmod, unit_type_tab.astype(f32).reshape(-1))
    return out.reshape(N_TOK, D)[None]


# speedup vs baseline: 5.2621x; 1.0217x over previous
import functools

import jax
import jax.numpy as jnp
from jax import lax
from jax.experimental import pallas as pl
from jax.experimental.pallas import tpu as pltpu
from jax.experimental.pallas import tpu_sc as plsc

N_HEX = 131072
N_UNIT = 32768
D = 128
N_TOK = N_HEX + N_UNIT + 2

NW = 32
NC = 2
HEX_PER_W = N_HEX // NW
UNIT_PER_W = N_UNIT // NW
HP = 1024
SUB = 64


def _sc_encode(hx, hy, htm, ut, ux, uy, P, gf, wg, et,
               px_t, py_t, tm_t, ty_t):
    mesh = plsc.VectorSubcoreMesh(core_axis_name="c", subcore_axis_name="s")

    @functools.partial(
        pl.kernel,
        out_type=jax.ShapeDtypeStruct((N_TOK * D,), jnp.float32),
        mesh=mesh,
        compiler_params=pltpu.CompilerParams(needs_layout_passes=False),
        scratch_types=[
            pltpu.VMEM((128 * D,), jnp.float32),
            pltpu.VMEM((128 * D,), jnp.float32),
            pltpu.VMEM((128 * D,), jnp.float32),
            pltpu.VMEM((200 * D,), jnp.float32),
            pltpu.VMEM((HP,), jnp.int32),
            pltpu.VMEM((HP,), jnp.int32),
            pltpu.VMEM((HP,), jnp.int32),
            pltpu.VMEM((UNIT_PER_W,), jnp.int32),
            pltpu.VMEM((UNIT_PER_W,), jnp.int32),
            pltpu.VMEM((UNIT_PER_W,), jnp.int32),
            pltpu.VMEM((SUB * D,), jnp.float32),
            pltpu.VMEM((SUB * D,), jnp.float32),
            pltpu.VMEM((SUB * D,), jnp.float32),
            pltpu.VMEM((SUB * D,), jnp.float32),
            pltpu.VMEM((8 * D,), jnp.float32),
            pltpu.VMEM((8 * D,), jnp.float32),
            pltpu.VMEM((2 * D,), jnp.float32),
            pltpu.SemaphoreType.DMA,
            pltpu.SemaphoreType.DMA,
            pltpu.SemaphoreType.DMA,
            pltpu.SemaphoreType.DMA,
        ],
    )
    def k(hx_h, hy_h, htm_h, ut_h, ux_h, uy_h, p_h,
          gf_h, wg_h, et_h,
          px_h, py_h, tm_h, ty_h, out_h,
          px_v, py_v, tm_v, ty_v, hx_v, hy_v, htm_v,
          ut_v, ux_v, uy_v, p0_v, p1_v, o0_v, o1_v, gf_v, wg_v, g2_v,
          so0, so1, sp0, sp1):
        wid = lax.axis_index("s") * NC + lax.axis_index("c")
        so = (so0, so1)
        sp = (sp0, sp1)
        ovs = (o0_v, o1_v)
        pvs = (p0_v, p1_v)

        pltpu.sync_copy(px_h, px_v)
        pltpu.sync_copy(py_h, py_v)
        pltpu.sync_copy(tm_h, tm_v)
        pltpu.sync_copy(ty_h, ty_v)

        hbase = wid * HEX_PER_W

        @pl.loop(0, HEX_PER_W // HP)
        def _piece(piece):
            pb = hbase + piece * HP
            pltpu.sync_copy(hx_h.at[pl.ds(pb, HP)], hx_v)
            pltpu.sync_copy(hy_h.at[pl.ds(pb, HP)], hy_v)
            pltpu.sync_copy(htm_h.at[pl.ds(pb, HP)], htm_v)

            @pl.loop(0, HP // SUB, step=2)
            def _sub(sub):
                for b in range(2):
                    s = sub + b
                    t = piece * (HP // SUB) + s

                    @pl.when(t >= 2)
                    def _():
                        pltpu.make_async_copy(
                            out_h.at[pl.ds(0, SUB * D)], ovs[b],
                            so[b]).wait()

                    @pl.loop(0, 4)
                    def _grp(g):
                        o = s * SUB + g * 16
                        xs = hx_v[pl.ds(o, 16)]
                        ys = hy_v[pl.ds(o, 16)]
                        tms = htm_v[pl.ds(o, 16)]
                        bx = xs * D
                        by = ys * D
                        bt = tms * D
                        ob = ovs[b]
                        for u in range(16):
                            xo = bx[u]
                            yo = by[u]
                            to = bt[u]
                            lo = (g * 16 + u) * D
                            rx = [px_v[pl.ds(xo + 16 * vi, 16)]
                                  for vi in range(8)]
                            ry = [py_v[pl.ds(yo + 16 * vi, 16)]
                                  for vi in range(8)]
                            rt = [tm_v[pl.ds(to + 16 * vi, 16)]
                                  for vi in range(8)]
                            for vi in range(8):
                                ob[pl.ds(lo + 16 * vi, 16)] = (
                                    rx[vi] + ry[vi] + rt[vi])

                    rowb = pb + s * SUB
                    pltpu.async_copy(ovs[b],
                                     out_h.at[pl.ds(rowb * D, SUB * D)],
                                     so[b])

        for b in range(2):
            pltpu.make_async_copy(out_h.at[pl.ds(0, SUB * D)], ovs[b],
                                  so[b]).wait()

        ubase = wid * UNIT_PER_W
        pltpu.sync_copy(ut_h.at[pl.ds(ubase, UNIT_PER_W)], ut_v)
        pltpu.sync_copy(ux_h.at[pl.ds(ubase, UNIT_PER_W)], ux_v)
        pltpu.sync_copy(uy_h.at[pl.ds(ubase, UNIT_PER_W)], uy_v)

        NSUB = UNIT_PER_W // SUB
        for b in range(2):
            pltpu.async_copy(p_h.at[pl.ds((ubase + b * SUB) * D, SUB * D)],
                             pvs[b], sp[b])

        @pl.loop(0, NSUB, step=2)
        def _usub(sub):
            for b in range(2):
                s = sub + b
                pltpu.make_async_copy(p_h.at[pl.ds(0, SUB * D)], pvs[b],
                                      sp[b]).wait()

                @pl.when(s >= 2)
                def _():
                    pltpu.make_async_copy(
                        out_h.at[pl.ds(0, SUB * D)], ovs[b], so[b]).wait()

                @pl.loop(0, 4)
                def _grp(g):
                    o = s * SUB + g * 16
                    tv = ut_v[pl.ds(o, 16)]
                    xv = ux_v[pl.ds(o, 16)]
                    yv = uy_v[pl.ds(o, 16)]
                    btv = tv * D
                    bxv = xv * D
                    byv = yv * D
                    ob = ovs[b]
                    pb2 = pvs[b]
                    for u in range(16):
                        to = btv[u]
                        xo = bxv[u]
                        yo = byv[u]
                        lo = (g * 16 + u) * D
                        rt = [ty_v[pl.ds(to + 16 * vi, 16)]
                              for vi in range(8)]
                        rx = [px_v[pl.ds(xo + 16 * vi, 16)]
                              for vi in range(8)]
                        ry = [py_v[pl.ds(yo + 16 * vi, 16)]
                              for vi in range(8)]
                        rp = [pb2[pl.ds(lo + 16 * vi, 16)]
                              for vi in range(8)]
                        for vi in range(8):
                            ob[pl.ds(lo + 16 * vi, 16)] = (
                                (rt[vi] + rx[vi]) + (ry[vi] + rp[vi]))

                rowb = ubase + s * SUB
                pltpu.async_copy(ovs[b],
                                 out_h.at[pl.ds((N_HEX + rowb) * D, SUB * D)],
                                 so[b])

                @pl.when(s + 2 < NSUB)
                def _():
                    pltpu.async_copy(
                        p_h.at[pl.ds((ubase + (s + 2) * SUB) * D, SUB * D)],
                        pvs[b], sp[b])

        for b in range(2):
            pltpu.make_async_copy(out_h.at[pl.ds(0, SUB * D)], ovs[b],
                                  so[b]).wait()

        @pl.when(wid == NW - 1)
        def _tail():
            pltpu.sync_copy(gf_h, gf_v)
            pltpu.sync_copy(wg_h, wg_v)
            pltpu.sync_copy(et_h, g2_v.at[pl.ds(D, D)])
            for dg in range(8):
                acc = jnp.zeros((16,), jnp.float32)
                for kk in range(8):
                    wkr = wg_v[pl.ds(kk * D + dg * 16, 16)]
                    gk = gf_v[pl.ds(kk * D + dg * 16, 16)]
                    acc = acc + gk * wkr
                g2_v[pl.ds(dg * 16, 16)] = acc
            pltpu.sync_copy(g2_v,
                            out_h.at[pl.ds((N_HEX + N_UNIT) * D, 2 * D)])

    return k(hx, hy, htm, ut, ux, uy, P, gf, wg, et,
             px_t, py_t, tm_t, ty_t)


def _tc_proj(X, W):
    PB = 2048

    def body(x_ref, w_ref, o_ref):
        o_ref[...] = jnp.dot(x_ref[...], w_ref[...],
                             preferred_element_type=jnp.float32)

    return pl.pallas_call(
        body,
        grid=(N_UNIT // PB,),
        in_specs=[
            pl.BlockSpec((PB, 16), lambda i: (i, 0)),
            pl.BlockSpec((16, D), lambda i: (0, 0)),
        ],
        out_specs=pl.BlockSpec((PB, D), lambda i: (i, 0)),
        out_shape=jax.ShapeDtypeStruct((N_UNIT, D), jnp.float32),
    )(X, W)


def kernel(hex_xs, hex_ys, terr_ids, mod_flags, unit_type_ids, unit_side_ids,
           unit_xs, unit_ys, unit_feats, global_feats, terrain_tab, Wmod,
           posx_tab, posy_tab, unit_type_tab, Wf, bf, side_tab, Wg, bg,
           end_turn):
    f32 = jnp.float32
    i32 = jnp.int32

    bits = ((jnp.arange(8)[:, None] >> jnp.array([2, 1, 0])[None, :]) & 1
            ).astype(f32)
    modcomb = bits @ Wmod
    terrmod = (terrain_tab[:, None, :] + modcomb[None, :, :]
               ).reshape(16 * 8 * D).astype(f32)

    mcode = mod_flags.astype(f32) @ jnp.array([4.0, 2.0, 1.0], f32)
    htm = terr_ids.astype(i32) * 8 + mcode.astype(i32)

    side01 = unit_side_ids.astype(f32)[:, None]
    ones = jnp.ones((N_UNIT, 1), f32)
    zeros = jnp.zeros((N_UNIT, 1), f32)
    X = jnp.concatenate([unit_feats.astype(f32), side01, ones, zeros],
                        axis=1)
    W = jnp.concatenate([
        Wf.astype(f32),
        (side_tab[1] - side_tab[0])[None, :],
        (bf + side_tab[0])[None, :],
        jnp.zeros((1, D), f32),
    ], axis=0)

    P = _tc_proj(X, W)

    gf1 = jnp.concatenate([global_feats[0].astype(f32),
                           jnp.ones((1,), f32), jnp.zeros((1,), f32)])
    gf = jnp.broadcast_to(gf1[:, None], (8, D)).reshape(-1)
    wg = jnp.concatenate([Wg.astype(f32), bg[None, :].astype(f32),
                          jnp.zeros((1, D), f32)], axis=0).reshape(-1)

    out = _sc_encode(
        hex_xs.astype(i32), hex_ys.astype(i32), htm,
        unit_type_ids.astype(i32), unit_xs.astype(i32), unit_ys.astype(i32),
        P.reshape(-1), gf, wg, end_turn.astype(f32),
        posx_tab.astype(f32).reshape(-1), posy_tab.astype(f32).reshape(-1),
        terrmod, unit_type_tab.astype(f32).reshape(-1))
    return out.reshape(N_TOK, D)[None]
